# baseline (speedup 1.0000x reference)
import functools

import jax
import jax.numpy as jnp
from jax.experimental import pallas as pl
from jax.experimental.pallas import tpu as pltpu

K_NEIGH = 60
N_PAD = 100352
BIG = 1e30


def _qmean_body(hs_ref, q_ref):
    q_ref[...] = jnp.mean(hs_ref[...], axis=1)


def _dist_body(q_ref, keys_ref, k2_ref, d2_ref):
    qk = jax.lax.dot_general(
        q_ref[...], keys_ref[...], (((1,), (1,)), ((), ())),
        preferred_element_type=jnp.float32)
    q2 = jnp.sum(q_ref[...] * q_ref[...], axis=1, keepdims=True)
    d2_ref[...] = (q2 + k2_ref[...]) - 2.0 * qk


def _mlp_body(kd_ref, ks_ref, p_ref, kw1_ref, kb1_ref, kw2_ref, kb2_ref,
              lw1_ref, lb1_ref, lw2_ref, lb2_ref, out_ref):
    kd = kd_ref[...]
    ks = ks_ref[...]
    p = p_ref[...]
    h = jnp.maximum(
        jax.lax.dot_general(kd, kw1_ref[...], (((1,), (0,)), ((), ())),
                            preferred_element_type=jnp.float32) + kb1_ref[...],
        0.0)
    logits = jax.lax.dot_general(h, kw2_ref[...], (((1,), (0,)), ((), ())),
                                 preferred_element_type=jnp.float32) + kb2_ref[...]
    logits = logits - jnp.max(logits, axis=1, keepdims=True)
    e = jnp.exp(logits)
    w = e / jnp.sum(e, axis=1, keepdims=True)
    np_s = jnp.sum(w * ks, axis=1, keepdims=True)
    lw1 = lw1_ref[...]
    lh = (p * lw1[0:1, :] + np_s * lw1[1:2, :]
          + jax.lax.dot_general(kd, lw1[2:, :], (((1,), (0,)), ((), ())),
                                preferred_element_type=jnp.float32)
          + lb1_ref[...])
    lh = jnp.maximum(lh, 0.0)
    ll = jax.lax.dot_general(lh, lw2_ref[...], (((1,), (0,)), ((), ())),
                             preferred_element_type=jnp.float32) + lb2_ref[...]
    ll = ll - jnp.max(ll, axis=1, keepdims=True)
    le = jnp.exp(ll)
    lam = le / jnp.sum(le, axis=1, keepdims=True)
    out_ref[...] = (p * lam[:, 0:1] + np_s * lam[:, 1:2])


def kernel(hs, p_scores, ds_keys, ds_scores, knet_w1, knet_b1, knet_w2,
           knet_b2, lam_w1, lam_b1, lam_w2, lam_b2):
    B, T, D = hs.shape
    N = ds_keys.shape[0]

    q = pl.pallas_call(
        _qmean_body,
        grid=(8,),
        in_specs=[pl.BlockSpec((B // 8, T, D), lambda i: (i, 0, 0))],
        out_specs=pl.BlockSpec((B // 8, D), lambda i: (i, 0)),
        out_shape=jax.ShapeDtypeStruct((B, D), jnp.float32),
    )(hs)

    keys_pad = jnp.concatenate(
        [ds_keys, jnp.zeros((N_PAD - N, D), jnp.float32)], axis=0)
    k2 = jnp.sum(keys_pad * keys_pad, axis=1)
    k2 = jnp.where(jnp.arange(N_PAD) < N, k2, BIG)[None, :]

    C = 2048
    d2 = pl.pallas_call(
        _dist_body,
        grid=(N_PAD // C,),
        in_specs=[
            pl.BlockSpec((B, D), lambda i: (0, 0)),
            pl.BlockSpec((C, D), lambda i: (i, 0)),
            pl.BlockSpec((1, C), lambda i: (0, i)),
        ],
        out_specs=pl.BlockSpec((B, C), lambda i: (0, i)),
        out_shape=jax.ShapeDtypeStruct((B, N_PAD), jnp.float32),
    )(q, keys_pad, k2)

    neg_d, idx = jax.lax.top_k(-d2, K_NEIGH)
    knn_d = -neg_d
    knn_s = jnp.take(ds_scores, idx, axis=0)

    out = pl.pallas_call(
        _mlp_body,
        grid=(1,),
        in_specs=[
            pl.BlockSpec((B, K_NEIGH), lambda i: (0, 0)),
            pl.BlockSpec((B, K_NEIGH), lambda i: (0, 0)),
            pl.BlockSpec((B, 1), lambda i: (0, 0)),
            pl.BlockSpec(knet_w1.shape, lambda i: (0, 0)),
            pl.BlockSpec((1, 128), lambda i: (0, 0)),
            pl.BlockSpec(knet_w2.shape, lambda i: (0, 0)),
            pl.BlockSpec((1, K_NEIGH), lambda i: (0, 0)),
            pl.BlockSpec(lam_w1.shape, lambda i: (0, 0)),
            pl.BlockSpec((1, 128), lambda i: (0, 0)),
            pl.BlockSpec(lam_w2.shape, lambda i: (0, 0)),
            pl.BlockSpec((1, 2), lambda i: (0, 0)),
        ],
        out_specs=pl.BlockSpec((B, 1), lambda i: (0, 0)),
        out_shape=jax.ShapeDtypeStruct((B, 1), jnp.float32),
    )(knn_d, knn_s, p_scores[:, None], knet_w1, knet_b1[None, :], knet_w2,
      knet_b2[None, :], lam_w1, lam_b1[None, :], lam_w2, lam_b2[None, :])
    return out


# speedup vs baseline: 16.8120x; 16.8120x over previous
import functools

import jax
import jax.numpy as jnp
from jax import lax
from jax.experimental import pallas as pl
from jax.experimental.pallas import tpu as pltpu
from jax.experimental.pallas import tpu_sc as plsc

K_NEIGH = 60
B = 1024
D = 32
N = 100000
CHUNK = 128
M_CHUNKS = 784
N_PAD = M_CHUNKS * CHUNK
C_BLK = 2048
N_BLKS = N_PAD // C_BLK
BIG = 1e30
KEY_INF = 3.0e38
NC, NS = 2, 16
NW = NC * NS
ROWS_PER_W = B // NW
CAND_CAP = 1024


def _qmean_body(hs_ref, q_ref):
    q_ref[...] = jnp.mean(hs_ref[...], axis=1)


def _dist_body(q_ref, keyst_ref, d2_ref, dmin_ref):
    i = pl.program_id(0)
    q = q_ref[...]
    kt = keyst_ref[...]
    qk = lax.dot_general(q, kt, (((1,), (0,)), ((), ())),
                         preferred_element_type=jnp.float32)
    q2 = jnp.sum(q * q, axis=1, keepdims=True)
    k2 = jnp.sum(kt * kt, axis=0, keepdims=True)
    d2 = (q2 + k2) - 2.0 * qk
    col = i * C_BLK + lax.broadcasted_iota(jnp.int32, (1, C_BLK), 1)
    d2 = jnp.where(col < N, d2, BIG)
    d2_ref[...] = d2
    dmin_ref[...] = jnp.min(d2.reshape(B, 16, CHUNK), axis=-1)[None]


def _merge64(kx, vx, bufs):
    b0k, b1k, b2k, b3k, b0v, b1v, b2v, b3v = bufs
    bk = [b0k, b1k, b2k, b3k]
    bv = [b0v, b1v, b2v, b3v]
    for i in range(4):
        rk = lax.rev(kx, (0,))
        rv = lax.rev(vx, (0,))
        m = bk[i] <= rk
        lok = jnp.where(m, bk[i], rk)
        lov = jnp.where(m, bv[i], rv)
        hik = jnp.where(m, rk, bk[i])
        hiv = jnp.where(m, rv, bv[i])
        bk[i], bv[i] = plsc.sort_key_val(lok, lov)
        kx, vx = plsc.sort_key_val(hik, hiv)
    return (bk[0], bk[1], bk[2], bk[3], bv[0], bv[1], bv[2], bv[3])


def _stream_step(kx, vx, carry):
    cnt = plsc.all_reduce_population_count(kx < carry[8])

    def do(c):
        skx, svx = plsc.sort_key_val(kx, vx)
        b = _merge64(skx, svx, c[:8])
        return b + (jnp.full((16,), b[3][15], jnp.float32),)

    return lax.cond(cnt[0] > 0, do, lambda c: c, carry)


def _fresh_bufs():
    k = jnp.full((16,), KEY_INF, jnp.float32)
    v = jnp.zeros((16,), jnp.int32)
    return (k, k, k, k, v, v, v, v, k)


def _sc_body(dmin_hbm, d2_hbm, sct_hbm, knnd_hbm, knns_hbm, *s):
    (dminb0, idxd0, idxs0, d2v0, scv0,
     dminb1, idxd1, idxs1, d2v1, scv1,
     candk_v, candv_v, kout_v, sout_v,
     semm0, semd0, sems0, semm1, semd1, sems1) = s
    sets = ((dminb0, idxd0, idxs0, d2v0, scv0, semm0, semd0, sems0),
            (dminb1, idxd1, idxs1, d2v1, scv1, semm1, semd1, sems1))
    iota = lax.iota(jnp.int32, 16)
    inf_v = jnp.full((16,), KEY_INF, jnp.float32)
    wid = lax.axis_index("s") * NC + lax.axis_index("c")
    row0 = wid * ROWS_PER_W

    def issue_dmin(st, row):
        return pltpu.async_copy(dmin_hbm.at[row], st[0], st[5])

    def phase1(st, row):
        dminb = st[0]

        def p1(j, bufs):
            kx = dminb[pl.ds(j * 16, 16)]
            vx = j * 16 + iota
            return _stream_step(kx, vx, bufs)

        bufs = lax.fori_loop(0, N_BLKS, p1, _fresh_bufs())
        thr = bufs[3][11]
        for i in range(4):
            cid = bufs[4 + i]
            st[1][pl.ds(i * 16, 16)] = row * M_CHUNKS + cid
            st[2][pl.ds(i * 16, 16)] = cid
        return thr

    def drain(st, r_prev, thr):
        pltpu.make_async_copy(d2_hbm.at[st[1]], st[3], st[6]).wait()
        d2v, scv = st[3], st[4]
        thrv = jnp.full((16,), thr, jnp.float32)

        def p2(sl, off):
            srow = jnp.full((16,), sl, jnp.int32)

            def g_step(off, g):
                kx = plsc.load_gather(d2v, [srow, g * 16 + iota])
                m = kx <= thrv
                plsc.store_compressed(candk_v.at[pl.ds(off, 16)], kx, mask=m)
                vx = sl * 128 + g * 16 + iota
                plsc.store_compressed(candv_v.at[pl.ds(off, 16)], vx, mask=m)
                cnt = plsc.all_reduce_population_count(m)
                return jnp.minimum(off + cnt[0], CAND_CAP - 16)

            for g in range(8):
                off = g_step(off, g)
            return off

        off = lax.fori_loop(0, 64, p2, 0)
        candk_v[pl.ds(off, 16)] = inf_v

        def p3(j, bufs):
            kx = candk_v[pl.ds(j * 16, 16)]
            vx = candv_v[pl.ds(j * 16, 16)]
            return _stream_step(kx, vx, bufs)

        bufs = lax.fori_loop(0, (off + 15) >> 4, p3, _fresh_bufs())

        pltpu.make_async_copy(sct_hbm.at[st[2]], st[4], st[7]).wait()
        for i in range(4):
            kout_v[pl.ds(r_prev * 64 + i * 16, 16)] = bufs[i]
            pos = bufs[4 + i]
            sout_v[pl.ds(r_prev * 64 + i * 16, 16)] = plsc.load_gather(
                scv, [pos >> 7, pos & 127])

    def body_row(r, thr_prev, cur, prv):
        row = row0 + r
        pltpu.make_async_copy(dmin_hbm.at[row], cur[0], cur[5]).wait()
        thr = phase1(cur, row)

        @pl.when(r < ROWS_PER_W - 1)
        def _():
            issue_dmin(prv, row + 1)

        @pl.when(r >= 1)
        def _():
            drain(prv, r - 1, thr_prev)

        pltpu.async_copy(d2_hbm.at[cur[1]], cur[3], cur[6])
        pltpu.async_copy(sct_hbm.at[cur[2]], cur[4], cur[7])
        return thr

    issue_dmin(sets[0], row0)

    def pair(rr, thr):
        thr = body_row(2 * rr, thr, sets[0], sets[1])
        thr = body_row(2 * rr + 1, thr, sets[1], sets[0])
        return thr

    thr = lax.fori_loop(0, ROWS_PER_W // 2, pair, jnp.float32(0))
    drain(sets[1], ROWS_PER_W - 1, thr)

    pltpu.sync_copy(kout_v, knnd_hbm.at[pl.ds(row0 * 64, ROWS_PER_W * 64)])
    pltpu.sync_copy(sout_v, knns_hbm.at[pl.ds(row0 * 64, ROWS_PER_W * 64)])


def _sc_topk(dmin_flat, d2_flat, sc_tbl):
    mesh = plsc.VectorSubcoreMesh(core_axis_name="c", subcore_axis_name="s",
                                  num_cores=NC, num_subcores=NS)

    def pset():
        return [
            pltpu.VMEM((M_CHUNKS,), jnp.float32),
            pltpu.VMEM((64,), jnp.int32),
            pltpu.VMEM((64,), jnp.int32),
            pltpu.VMEM((64, CHUNK), jnp.float32),
            pltpu.VMEM((64, CHUNK), jnp.float32),
        ]

    f = pl.kernel(
        _sc_body,
        out_type=(jax.ShapeDtypeStruct((B * 64,), jnp.float32),
                  jax.ShapeDtypeStruct((B * 64,), jnp.float32)),
        mesh=mesh,
        compiler_params=pltpu.CompilerParams(needs_layout_passes=False),
        scratch_types=pset() + pset() + [
            pltpu.VMEM((CAND_CAP,), jnp.float32),
            pltpu.VMEM((CAND_CAP,), jnp.int32),
            pltpu.VMEM((ROWS_PER_W * 64,), jnp.float32),
            pltpu.VMEM((ROWS_PER_W * 64,), jnp.float32),
        ] + [pltpu.SemaphoreType.DMA] * 6,
    )
    return f(dmin_flat, d2_flat, sc_tbl)


def _mlp_body(kd_ref, ks_ref, p_ref, kw1_ref, kb1_ref, kw2_ref, kb2_ref,
              lw1_ref, lb1_ref, lw2_ref, lb2_ref, out_ref):
    kd = kd_ref[...][:, :K_NEIGH]
    ks = ks_ref[...][:, :K_NEIGH]
    p = p_ref[...]
    h = jnp.maximum(
        lax.dot_general(kd, kw1_ref[...], (((1,), (0,)), ((), ())),
                        preferred_element_type=jnp.float32) + kb1_ref[...],
        0.0)
    logits = lax.dot_general(h, kw2_ref[...], (((1,), (0,)), ((), ())),
                             preferred_element_type=jnp.float32) + kb2_ref[...]
    logits = logits - jnp.max(logits, axis=1, keepdims=True)
    e = jnp.exp(logits)
    w = e / jnp.sum(e, axis=1, keepdims=True)
    np_s = jnp.sum(w * ks, axis=1, keepdims=True)
    lw1 = lw1_ref[...]
    lh = (p * lw1[0:1, :] + np_s * lw1[1:2, :]
          + lax.dot_general(kd, lw1[2:, :], (((1,), (0,)), ((), ())),
                            preferred_element_type=jnp.float32)
          + lb1_ref[...])
    lh = jnp.maximum(lh, 0.0)
    ll = lax.dot_general(lh, lw2_ref[...], (((1,), (0,)), ((), ())),
                         preferred_element_type=jnp.float32) + lb2_ref[...]
    ll = ll - jnp.max(ll, axis=1, keepdims=True)
    le = jnp.exp(ll)
    lam = le / jnp.sum(le, axis=1, keepdims=True)
    out_ref[...] = (p * lam[:, 0:1] + np_s * lam[:, 1:2])


def kernel(hs, p_scores, ds_keys, ds_scores, knet_w1, knet_b1, knet_w2,
           knet_b2, lam_w1, lam_b1, lam_w2, lam_b2):
    T = hs.shape[1]

    q = pl.pallas_call(
        _qmean_body,
        grid=(8,),
        in_specs=[pl.BlockSpec((B // 8, T, D), lambda i: (i, 0, 0))],
        out_specs=pl.BlockSpec((B // 8, D), lambda i: (i, 0)),
        out_shape=jax.ShapeDtypeStruct((B, D), jnp.float32),
    )(hs)

    keys_t = ds_keys.T
    d2, dmin3 = pl.pallas_call(
        _dist_body,
        grid=(N_BLKS,),
        in_specs=[
            pl.BlockSpec((B, D), lambda i: (0, 0)),
            pl.BlockSpec((D, C_BLK), lambda i: (0, i)),
        ],
        out_specs=[
            pl.BlockSpec((B, C_BLK), lambda i: (0, i)),
            pl.BlockSpec((1, B, 16), lambda i: (i, 0, 0)),
        ],
        out_shape=[
            jax.ShapeDtypeStruct((B, N_PAD), jnp.float32),
            jax.ShapeDtypeStruct((N_BLKS, B, 16), jnp.float32),
        ],
    )(q, keys_t)

    dmin_flat = jnp.transpose(dmin3, (1, 0, 2)).reshape(B, M_CHUNKS)
    d2_flat = d2.reshape(B * M_CHUNKS, CHUNK)
    sc_tbl = jnp.concatenate(
        [ds_scores, jnp.zeros((N_PAD - N,), jnp.float32)]).reshape(
            M_CHUNKS, CHUNK)

    knnd64, knns64 = _sc_topk(dmin_flat, d2_flat, sc_tbl)
    knn_d = knnd64.reshape(B, 64)
    knn_s = knns64.reshape(B, 64)

    out = pl.pallas_call(
        _mlp_body,
        grid=(1,),
        in_specs=[
            pl.BlockSpec((B, 64), lambda i: (0, 0)),
            pl.BlockSpec((B, 64), lambda i: (0, 0)),
            pl.BlockSpec((B, 1), lambda i: (0, 0)),
            pl.BlockSpec(knet_w1.shape, lambda i: (0, 0)),
            pl.BlockSpec((1, 128), lambda i: (0, 0)),
            pl.BlockSpec(knet_w2.shape, lambda i: (0, 0)),
            pl.BlockSpec((1, K_NEIGH), lambda i: (0, 0)),
            pl.BlockSpec(lam_w1.shape, lambda i: (0, 0)),
            pl.BlockSpec((1, 128), lambda i: (0, 0)),
            pl.BlockSpec(lam_w2.shape, lambda i: (0, 0)),
            pl.BlockSpec((1, 2), lambda i: (0, 0)),
        ],
        out_specs=pl.BlockSpec((B, 1), lambda i: (0, 0)),
        out_shape=jax.ShapeDtypeStruct((B, 1), jnp.float32),
    )(knn_d, knn_s, p_scores[:, None], knet_w1, knet_b1[None, :], knet_w2,
      knet_b2[None, :], lam_w1, lam_b1[None, :], lam_w2, lam_b2[None, :])
    return out


# speedup vs baseline: 24.1152x; 1.4344x over previous
import functools

import jax
import jax.numpy as jnp
from jax import lax
from jax.experimental import pallas as pl
from jax.experimental.pallas import tpu as pltpu
from jax.experimental.pallas import tpu_sc as plsc

K_NEIGH = 60
B = 1024
D = 32
N = 100000
CHUNK = 128
M_CHUNKS = 784
N_PAD = M_CHUNKS * CHUNK
C_BLK = 2048
N_BLKS = N_PAD // C_BLK
BIG = 1e30
KEY_INF = 3.0e38
NC, NS = 2, 16
NW = NC * NS
ROWS_PER_W = B // NW
CAND_CAP = 1024


def _qmean_body(hs_ref, q_ref):
    q_ref[...] = jnp.mean(hs_ref[...], axis=1)


def _dist_body(q_ref, keyst_ref, d2_ref, dmin_ref):
    i = pl.program_id(0)
    q = q_ref[...]
    kt = keyst_ref[...]
    qk = lax.dot_general(q, kt, (((1,), (0,)), ((), ())),
                         preferred_element_type=jnp.float32)
    q2 = jnp.sum(q * q, axis=1, keepdims=True)
    k2 = jnp.sum(kt * kt, axis=0, keepdims=True)
    d2 = (q2 + k2) - 2.0 * qk
    col = i * C_BLK + lax.broadcasted_iota(jnp.int32, (1, C_BLK), 1)
    d2 = jnp.where(col < N, d2, BIG)
    for c in range(16):
        d2_ref[pl.ds(c * B, B), :] = d2[:, c * CHUNK:(c + 1) * CHUNK]
    dmin_ref[...] = jnp.min(d2.reshape(B, 16, CHUNK), axis=-1)[None]


def _merge64(kx, vx, bufs):
    b0k, b1k, b2k, b3k, b0v, b1v, b2v, b3v = bufs
    bk = [b0k, b1k, b2k, b3k]
    bv = [b0v, b1v, b2v, b3v]
    for i in range(4):
        rk = lax.rev(kx, (0,))
        rv = lax.rev(vx, (0,))
        m = bk[i] <= rk
        lok = jnp.where(m, bk[i], rk)
        lov = jnp.where(m, bv[i], rv)
        hik = jnp.where(m, rk, bk[i])
        hiv = jnp.where(m, rv, bv[i])
        bk[i], bv[i] = plsc.sort_key_val(lok, lov)
        kx, vx = plsc.sort_key_val(hik, hiv)
    return (bk[0], bk[1], bk[2], bk[3], bv[0], bv[1], bv[2], bv[3])


def _stream_step(kx, vx, carry):
    cnt = plsc.all_reduce_population_count(kx < carry[8])

    def do(c):
        skx, svx = plsc.sort_key_val(kx, vx)
        b = _merge64(skx, svx, c[:8])
        return b + (jnp.full((16,), b[3][15], jnp.float32),)

    return lax.cond(cnt[0] > 0, do, lambda c: c, carry)


def _fresh_bufs():
    k = jnp.full((16,), KEY_INF, jnp.float32)
    v = jnp.zeros((16,), jnp.int32)
    return (k, k, k, k, v, v, v, v, k)


def _sc_body(dmin_hbm, d2_hbm, sct_hbm, knnd_hbm, knns_hbm, *s):
    (dminb0, idxd0, idxs0, d2v0, scv0,
     dminb1, idxd1, idxs1, d2v1, scv1,
     candk_v, candv_v, kout_v, sout_v,
     semm0, semd0, sems0, semm1, semd1, sems1) = s
    sets = ((dminb0, idxd0, idxs0, d2v0, scv0, semm0, semd0, sems0),
            (dminb1, idxd1, idxs1, d2v1, scv1, semm1, semd1, sems1))
    iota = lax.iota(jnp.int32, 16)
    inf_v = jnp.full((16,), KEY_INF, jnp.float32)
    wid = lax.axis_index("s") * NC + lax.axis_index("c")
    row0 = wid * ROWS_PER_W

    def issue_dmin(st, row):
        return pltpu.async_copy(dmin_hbm.at[row], st[0], st[5])

    def phase1(st, row):
        dminb = st[0]

        def p1(j, bufs):
            kx = dminb[pl.ds(j * 16, 16)]
            vx = j * 16 + iota
            return _stream_step(kx, vx, bufs)

        bufs = lax.fori_loop(0, N_BLKS, p1, _fresh_bufs())
        thr = bufs[3][11]
        for i in range(4):
            cid = bufs[4 + i]
            st[1][pl.ds(i * 16, 16)] = cid * B + row
            st[2][pl.ds(i * 16, 16)] = cid
        return thr

    def drain(st, r_prev, thr):
        pltpu.make_async_copy(d2_hbm.at[st[1]], st[3], st[6]).wait()
        d2v, scv = st[3], st[4]
        thrv = jnp.full((16,), thr, jnp.float32)

        def p2(sl, off):
            srow = jnp.full((16,), sl, jnp.int32)

            def g_step(off, g):
                kx = plsc.load_gather(d2v, [srow, g * 16 + iota])
                m = kx <= thrv
                plsc.store_compressed(candk_v.at[pl.ds(off, 16)], kx, mask=m)
                vx = sl * 128 + g * 16 + iota
                plsc.store_compressed(candv_v.at[pl.ds(off, 16)], vx, mask=m)
                cnt = plsc.all_reduce_population_count(m)
                return jnp.minimum(off + cnt[0], CAND_CAP - 16)

            for g in range(8):
                off = g_step(off, g)
            return off

        off = lax.fori_loop(0, 64, p2, 0)
        candk_v[pl.ds(off, 16)] = inf_v

        def p3(j, bufs):
            kx = candk_v[pl.ds(j * 16, 16)]
            vx = candv_v[pl.ds(j * 16, 16)]
            return _stream_step(kx, vx, bufs)

        bufs = lax.fori_loop(0, (off + 15) >> 4, p3, _fresh_bufs())

        pltpu.make_async_copy(sct_hbm.at[st[2]], st[4], st[7]).wait()
        for i in range(4):
            kout_v[pl.ds(r_prev * 64 + i * 16, 16)] = bufs[i]
            pos = bufs[4 + i]
            sout_v[pl.ds(r_prev * 64 + i * 16, 16)] = plsc.load_gather(
                scv, [pos >> 7, pos & 127])

    def body_row(r, thr_prev, cur, prv):
        row = row0 + r
        pltpu.make_async_copy(dmin_hbm.at[row], cur[0], cur[5]).wait()
        thr = phase1(cur, row)

        @pl.when(r < ROWS_PER_W - 1)
        def _():
            issue_dmin(prv, row + 1)

        @pl.when(r >= 1)
        def _():
            drain(prv, r - 1, thr_prev)

        pltpu.async_copy(d2_hbm.at[cur[1]], cur[3], cur[6])
        pltpu.async_copy(sct_hbm.at[cur[2]], cur[4], cur[7])
        return thr

    issue_dmin(sets[0], row0)

    def pair(rr, thr):
        thr = body_row(2 * rr, thr, sets[0], sets[1])
        thr = body_row(2 * rr + 1, thr, sets[1], sets[0])
        return thr

    thr = lax.fori_loop(0, ROWS_PER_W // 2, pair, jnp.float32(0))
    drain(sets[1], ROWS_PER_W - 1, thr)

    pltpu.sync_copy(kout_v, knnd_hbm.at[pl.ds(row0 * 64, ROWS_PER_W * 64)])
    pltpu.sync_copy(sout_v, knns_hbm.at[pl.ds(row0 * 64, ROWS_PER_W * 64)])


def _sc_topk(dmin_flat, d2_flat, sc_tbl):
    mesh = plsc.VectorSubcoreMesh(core_axis_name="c", subcore_axis_name="s",
                                  num_cores=NC, num_subcores=NS)

    def pset():
        return [
            pltpu.VMEM((M_CHUNKS,), jnp.float32),
            pltpu.VMEM((64,), jnp.int32),
            pltpu.VMEM((64,), jnp.int32),
            pltpu.VMEM((64, CHUNK), jnp.float32),
            pltpu.VMEM((64, CHUNK), jnp.float32),
        ]

    f = pl.kernel(
        _sc_body,
        out_type=(jax.ShapeDtypeStruct((B * 64,), jnp.float32),
                  jax.ShapeDtypeStruct((B * 64,), jnp.float32)),
        mesh=mesh,
        compiler_params=pltpu.CompilerParams(needs_layout_passes=False),
        scratch_types=pset() + pset() + [
            pltpu.VMEM((CAND_CAP,), jnp.float32),
            pltpu.VMEM((CAND_CAP,), jnp.int32),
            pltpu.VMEM((ROWS_PER_W * 64,), jnp.float32),
            pltpu.VMEM((ROWS_PER_W * 64,), jnp.float32),
        ] + [pltpu.SemaphoreType.DMA] * 6,
    )
    return f(dmin_flat, d2_flat, sc_tbl)


def _mlp_body(kd_ref, ks_ref, p_ref, kw1_ref, kb1_ref, kw2_ref, kb2_ref,
              lw1_ref, lb1_ref, lw2_ref, lb2_ref, out_ref):
    kd = kd_ref[...][:, :K_NEIGH]
    ks = ks_ref[...][:, :K_NEIGH]
    p = p_ref[...]
    h = jnp.maximum(
        lax.dot_general(kd, kw1_ref[...], (((1,), (0,)), ((), ())),
                        preferred_element_type=jnp.float32) + kb1_ref[...],
        0.0)
    logits = lax.dot_general(h, kw2_ref[...], (((1,), (0,)), ((), ())),
                             preferred_element_type=jnp.float32) + kb2_ref[...]
    logits = logits - jnp.max(logits, axis=1, keepdims=True)
    e = jnp.exp(logits)
    w = e / jnp.sum(e, axis=1, keepdims=True)
    np_s = jnp.sum(w * ks, axis=1, keepdims=True)
    lw1 = lw1_ref[...]
    lh = (p * lw1[0:1, :] + np_s * lw1[1:2, :]
          + lax.dot_general(kd, lw1[2:, :], (((1,), (0,)), ((), ())),
                            preferred_element_type=jnp.float32)
          + lb1_ref[...])
    lh = jnp.maximum(lh, 0.0)
    ll = lax.dot_general(lh, lw2_ref[...], (((1,), (0,)), ((), ())),
                         preferred_element_type=jnp.float32) + lb2_ref[...]
    ll = ll - jnp.max(ll, axis=1, keepdims=True)
    le = jnp.exp(ll)
    lam = le / jnp.sum(le, axis=1, keepdims=True)
    out_ref[...] = (p * lam[:, 0:1] + np_s * lam[:, 1:2])


def kernel(hs, p_scores, ds_keys, ds_scores, knet_w1, knet_b1, knet_w2,
           knet_b2, lam_w1, lam_b1, lam_w2, lam_b2):
    T = hs.shape[1]

    q = pl.pallas_call(
        _qmean_body,
        grid=(8,),
        in_specs=[pl.BlockSpec((B // 8, T, D), lambda i: (i, 0, 0))],
        out_specs=pl.BlockSpec((B // 8, D), lambda i: (i, 0)),
        out_shape=jax.ShapeDtypeStruct((B, D), jnp.float32),
    )(hs)

    keys_t = ds_keys.T
    d2, dmin3 = pl.pallas_call(
        _dist_body,
        grid=(N_BLKS,),
        in_specs=[
            pl.BlockSpec((B, D), lambda i: (0, 0)),
            pl.BlockSpec((D, C_BLK), lambda i: (0, i)),
        ],
        out_specs=[
            pl.BlockSpec((16 * B, CHUNK), lambda i: (i, 0)),
            pl.BlockSpec((1, B, 16), lambda i: (i, 0, 0)),
        ],
        out_shape=[
            jax.ShapeDtypeStruct((M_CHUNKS * B, CHUNK), jnp.float32),
            jax.ShapeDtypeStruct((N_BLKS, B, 16), jnp.float32),
        ],
    )(q, keys_t)

    dmin_flat = jnp.transpose(dmin3, (1, 0, 2)).reshape(B, M_CHUNKS)
    d2_flat = d2
    sc_tbl = jnp.concatenate(
        [ds_scores, jnp.zeros((N_PAD - N,), jnp.float32)]).reshape(
            M_CHUNKS, CHUNK)

    knnd64, knns64 = _sc_topk(dmin_flat, d2_flat, sc_tbl)
    knn_d = knnd64.reshape(B, 64)
    knn_s = knns64.reshape(B, 64)

    out = pl.pallas_call(
        _mlp_body,
        grid=(1,),
        in_specs=[
            pl.BlockSpec((B, 64), lambda i: (0, 0)),
            pl.BlockSpec((B, 64), lambda i: (0, 0)),
            pl.BlockSpec((B, 1), lambda i: (0, 0)),
            pl.BlockSpec(knet_w1.shape, lambda i: (0, 0)),
            pl.BlockSpec((1, 128), lambda i: (0, 0)),
            pl.BlockSpec(knet_w2.shape, lambda i: (0, 0)),
            pl.BlockSpec((1, K_NEIGH), lambda i: (0, 0)),
            pl.BlockSpec(lam_w1.shape, lambda i: (0, 0)),
            pl.BlockSpec((1, 128), lambda i: (0, 0)),
            pl.BlockSpec(lam_w2.shape, lambda i: (0, 0)),
            pl.BlockSpec((1, 2), lambda i: (0, 0)),
        ],
        out_specs=pl.BlockSpec((B, 1), lambda i: (0, 0)),
        out_shape=jax.ShapeDtypeStruct((B, 1), jnp.float32),
    )(knn_d, knn_s, p_scores[:, None], knet_w1, knet_b1[None, :], knet_w2,
      knet_b2[None, :], lam_w1, lam_b1[None, :], lam_w2, lam_b2[None, :])
    return out
